# Initial kernel scaffold; baseline (speedup 1.0000x reference)
import functools

import jax
import jax.numpy as jnp
from jax import lax
from jax.experimental import pallas as pl
from jax.experimental.pallas import tpu as pltpu
from jax.experimental.pallas import tpu_sc as plsc

_N = 10000
_E = 320000
_D = 128
_L = 3

_CH = 100
_NW = 32
_EPW = _E // _NW
_CHUNKS = _EPW // _CH
_ZROWS = _N // 16
_ZCH = 125

_HIGH = jax.lax.Precision.HIGHEST


def _mp_sc(h, edge_attr, src2, dst2):
    mesh = plsc.VectorSubcoreMesh(core_axis_name="c", subcore_axis_name="s")

    @functools.partial(
        pl.kernel,
        out_type=jax.ShapeDtypeStruct((2, _N, _D), jnp.float32),
        mesh=mesh,
        scratch_types=[
            pltpu.VMEM_SHARED((_N, _D), jnp.float32),
            pltpu.VMEM((1, _CH), jnp.int32),
            pltpu.VMEM((1, _CH), jnp.int32),
            pltpu.VMEM((_CH, _D), jnp.float32),
            pltpu.VMEM((_CH, _D), jnp.float32),
            pltpu.VMEM((_ZCH, _D), jnp.float32),
            pltpu.SemaphoreType.DMA,
        ],
    )
    def k(h_hbm, ea_hbm, src_hbm, dst_hbm, out_hbm,
          acc, srcv, dstv, rows, eav, zbuf, sem):
        cid = lax.axis_index("c")
        sid = lax.axis_index("s")
        wid = cid * 16 + sid

        zero = jnp.zeros((16,), jnp.float32)

        @pl.loop(0, _ZCH)
        def _(i):
            for cc in range(_D // 16):
                zbuf[i, pl.ds(cc * 16, 16)] = zero

        for t in range(_ZROWS // _ZCH):
            pltpu.sync_copy(zbuf, acc.at[pl.ds(sid * _ZROWS + t * _ZCH, _ZCH)])
        plsc.subcore_barrier()

        @pl.loop(0, _CHUNKS)
        def _(kk):
            r = wid * _CHUNKS + kk
            pltpu.sync_copy(src_hbm.at[pl.ds(r, 1)], srcv)
            pltpu.sync_copy(dst_hbm.at[pl.ds(r, 1)], dstv)
            pltpu.async_copy(h_hbm.at[srcv.at[0]], rows, sem).wait()
            pltpu.sync_copy(ea_hbm.at[pl.ds(r * _CH, _CH)], eav)

            @pl.loop(0, _CH)
            def _(i):
                for cc in range(_D // 16):
                    sl = pl.ds(cc * 16, 16)
                    rows[i, sl] = jnp.maximum(rows[i, sl] + eav[i, sl], 0.0)

            pltpu.sync_copy(rows, acc.at[dstv.at[0]], add=True)

        plsc.subcore_barrier()
        for t in range(_ZROWS // _ZCH):
            base = sid * _ZROWS + t * _ZCH
            pltpu.sync_copy(acc.at[pl.ds(base, _ZCH)],
                            out_hbm.at[cid, pl.ds(base, _ZCH)])

    return k(h, edge_attr, src2, dst2)


_BLK = 400


def _pre_body(x_ref, w_ref, b_ref, o_ref):
    y = jnp.dot(x_ref[...], w_ref[...],
                preferred_element_type=jnp.float32, precision=_HIGH)
    o_ref[...] = jnp.maximum(y + b_ref[...], 0.0)


def _tc_pre(x, w, b):
    return pl.pallas_call(
        _pre_body,
        grid=(_N // _BLK,),
        in_specs=[
            pl.BlockSpec((_BLK, _D), lambda i: (i, 0)),
            pl.BlockSpec((_D, _D), lambda i: (0, 0)),
            pl.BlockSpec((1, _D), lambda i: (0, 0)),
        ],
        out_specs=pl.BlockSpec((_BLK, _D), lambda i: (i, 0)),
        out_shape=jax.ShapeDtypeStruct((_N, _D), jnp.float32),
    )(x, w, b.reshape(1, _D))


def _head_body(x_ref, w_ref, b_ref, o_ref):
    y = jnp.dot(x_ref[...], w_ref[...],
                preferred_element_type=jnp.float32, precision=_HIGH)
    o_ref[...] = y + b_ref[...]


def _tc_head(x, w, b):
    d_out = w.shape[1]
    return pl.pallas_call(
        _head_body,
        grid=(_N // _BLK,),
        in_specs=[
            pl.BlockSpec((_BLK, _D), lambda i: (i, 0)),
            pl.BlockSpec((_D, d_out), lambda i: (0, 0)),
            pl.BlockSpec((1, d_out), lambda i: (0, 0)),
        ],
        out_specs=pl.BlockSpec((_BLK, d_out), lambda i: (i, 0)),
        out_shape=jax.ShapeDtypeStruct((_N, d_out), jnp.float32),
    )(x, w, b.reshape(1, d_out))


def _upd_body(h_ref, a_ref, w1_ref, b1_ref, w2_ref, b2_ref, sc_ref,
              g_ref, bb_ref, o_ref):
    h = h_ref[...]
    agg = a_ref[0] + a_ref[1]
    z = sc_ref[0, 0] * h + agg
    z = jnp.maximum(
        jnp.dot(z, w1_ref[...], preferred_element_type=jnp.float32,
                precision=_HIGH) + b1_ref[...], 0.0)
    z = jnp.dot(z, w2_ref[...], preferred_element_type=jnp.float32,
                precision=_HIGH) + b2_ref[...]
    hn = h + z
    mu = jnp.mean(hn, axis=-1, keepdims=True)
    var = jnp.mean((hn - mu) ** 2, axis=-1, keepdims=True)
    o_ref[...] = g_ref[...] * (hn - mu) / jnp.sqrt(var + 1e-5) + bb_ref[...]


def _tc_update(h, agg, w1, b1, w2, b2, scale, g, bb):
    return pl.pallas_call(
        _upd_body,
        grid=(_N // _BLK,),
        in_specs=[
            pl.BlockSpec((_BLK, _D), lambda i: (i, 0)),
            pl.BlockSpec((2, _BLK, _D), lambda i: (0, i, 0)),
            pl.BlockSpec((_D, _D), lambda i: (0, 0)),
            pl.BlockSpec((1, _D), lambda i: (0, 0)),
            pl.BlockSpec((_D, _D), lambda i: (0, 0)),
            pl.BlockSpec((1, _D), lambda i: (0, 0)),
            pl.BlockSpec((1, 1), lambda i: (0, 0)),
            pl.BlockSpec((1, _D), lambda i: (0, 0)),
            pl.BlockSpec((1, _D), lambda i: (0, 0)),
        ],
        out_specs=pl.BlockSpec((_BLK, _D), lambda i: (i, 0)),
        out_shape=jax.ShapeDtypeStruct((_N, _D), jnp.float32),
    )(h, agg, w1, b1.reshape(1, _D), w2, b2.reshape(1, _D),
      scale.reshape(1, 1), g.reshape(1, _D), bb.reshape(1, _D))


def kernel(x, edge_index, edge_attr, W_pre, b_pre, W1, b1, W2, b2, eps,
           ln_g, ln_b, W_head, b_head):
    src2 = edge_index[0].reshape(_E // _CH, _CH)
    dst2 = edge_index[1].reshape(_E // _CH, _CH)

    h = _tc_pre(x, W_pre, b_pre)
    for l in range(_L):
        agg = _mp_sc(h, edge_attr, src2, dst2)
        h = _tc_update(h, agg, W1[l], b1[l], W2[l], b2[l],
                       1.0 + eps[l], ln_g[l], ln_b[l])
    return _tc_head(h, W_head, b_head)


# speedup vs baseline: 2.9049x; 2.9049x over previous
import functools

import jax
import jax.numpy as jnp
from jax import lax
from jax.experimental import pallas as pl
from jax.experimental.pallas import tpu as pltpu
from jax.experimental.pallas import tpu_sc as plsc

_N = 10000
_E = 320000
_D = 128
_L = 3

_CH = 80
_NW = 32
_EPW = _E // _NW
_CHUNKS = _EPW // _CH
_NPAD = 10240
_ZROWS = _NPAD // 16
_ZCH = 128

_HIGH = jax.lax.Precision.HIGHEST


def _mp_sc(h, edge_attr, src1, dst1):
    mesh = plsc.VectorSubcoreMesh(core_axis_name="c", subcore_axis_name="s")

    @functools.partial(
        pl.kernel,
        out_type=jax.ShapeDtypeStruct((2, _N, _D), jnp.float32),
        mesh=mesh,
        scratch_types=[
            pltpu.VMEM_SHARED((_NPAD, _D), jnp.float32),
            pltpu.VMEM((_CH,), jnp.int32),
            pltpu.VMEM((_CH,), jnp.int32),
            pltpu.VMEM((_CH, _D), jnp.float32),
            pltpu.VMEM((_CH, _D), jnp.float32),
            pltpu.VMEM((_ZCH, _D), jnp.float32),
            pltpu.SemaphoreType.DMA,
        ],
    )
    def k(h_hbm, ea_hbm, src_hbm, dst_hbm, out_hbm,
          acc, srcv, dstv, rows, eav, zbuf, sem):
        cid = lax.axis_index("c")
        sid = lax.axis_index("s")
        wid = cid * 16 + sid

        zero = jnp.zeros((16,), jnp.float32)

        @pl.loop(0, _ZCH)
        def _(i):
            for cc in range(_D // 16):
                zbuf[i, pl.ds(cc * 16, 16)] = zero

        for t in range(_ZROWS // _ZCH):
            pltpu.sync_copy(zbuf, acc.at[pl.ds(sid * _ZROWS + t * _ZCH, _ZCH)])
        plsc.subcore_barrier()

        @pl.loop(0, _CHUNKS)
        def _(kk):
            base = wid * _EPW + kk * _CH
            pltpu.sync_copy(src_hbm.at[pl.ds(base, _CH)], srcv)
            pltpu.sync_copy(dst_hbm.at[pl.ds(base, _CH)], dstv)
            pltpu.async_copy(h_hbm.at[srcv], rows, sem).wait()
            pltpu.sync_copy(ea_hbm.at[pl.ds(base, _CH)], eav)

            @pl.loop(0, _CH)
            def _(i):
                for cc in range(_D // 16):
                    sl = pl.ds(cc * 16, 16)
                    rows[i, sl] = jnp.maximum(rows[i, sl] + eav[i, sl], 0.0)

            pltpu.sync_copy(rows, acc.at[dstv], add=True)

        plsc.subcore_barrier()

        @pl.when(sid < 15)
        def _():
            pltpu.sync_copy(acc.at[pl.ds(sid * _ZROWS, _ZROWS)],
                            out_hbm.at[cid, pl.ds(sid * _ZROWS, _ZROWS)])

        @pl.when(sid == 15)
        def _():
            pltpu.sync_copy(acc.at[pl.ds(15 * _ZROWS, _N - 15 * _ZROWS)],
                            out_hbm.at[cid, pl.ds(15 * _ZROWS, _N - 15 * _ZROWS)])

    return k(h, edge_attr, src1, dst1)


_BLK = 400


def _pre_body(x_ref, w_ref, b_ref, o_ref):
    y = jnp.dot(x_ref[...], w_ref[...],
                preferred_element_type=jnp.float32, precision=_HIGH)
    o_ref[...] = jnp.maximum(y + b_ref[...], 0.0)


def _tc_pre(x, w, b):
    return pl.pallas_call(
        _pre_body,
        grid=(_N // _BLK,),
        in_specs=[
            pl.BlockSpec((_BLK, _D), lambda i: (i, 0)),
            pl.BlockSpec((_D, _D), lambda i: (0, 0)),
            pl.BlockSpec((1, _D), lambda i: (0, 0)),
        ],
        out_specs=pl.BlockSpec((_BLK, _D), lambda i: (i, 0)),
        out_shape=jax.ShapeDtypeStruct((_N, _D), jnp.float32),
    )(x, w, b.reshape(1, _D))


def _head_body(x_ref, w_ref, b_ref, o_ref):
    y = jnp.dot(x_ref[...], w_ref[...],
                preferred_element_type=jnp.float32, precision=_HIGH)
    o_ref[...] = y + b_ref[...]


def _tc_head(x, w, b):
    d_out = w.shape[1]
    return pl.pallas_call(
        _head_body,
        grid=(_N // _BLK,),
        in_specs=[
            pl.BlockSpec((_BLK, _D), lambda i: (i, 0)),
            pl.BlockSpec((_D, d_out), lambda i: (0, 0)),
            pl.BlockSpec((1, d_out), lambda i: (0, 0)),
        ],
        out_specs=pl.BlockSpec((_BLK, d_out), lambda i: (i, 0)),
        out_shape=jax.ShapeDtypeStruct((_N, d_out), jnp.float32),
    )(x, w, b.reshape(1, d_out))


def _upd_body(h_ref, a_ref, w1_ref, b1_ref, w2_ref, b2_ref, sc_ref,
              g_ref, bb_ref, o_ref):
    h = h_ref[...]
    agg = a_ref[0] + a_ref[1]
    z = sc_ref[0, 0] * h + agg
    z = jnp.maximum(
        jnp.dot(z, w1_ref[...], preferred_element_type=jnp.float32,
                precision=_HIGH) + b1_ref[...], 0.0)
    z = jnp.dot(z, w2_ref[...], preferred_element_type=jnp.float32,
                precision=_HIGH) + b2_ref[...]
    hn = h + z
    mu = jnp.mean(hn, axis=-1, keepdims=True)
    var = jnp.mean((hn - mu) ** 2, axis=-1, keepdims=True)
    o_ref[...] = g_ref[...] * (hn - mu) / jnp.sqrt(var + 1e-5) + bb_ref[...]


def _tc_update(h, agg, w1, b1, w2, b2, scale, g, bb):
    return pl.pallas_call(
        _upd_body,
        grid=(_N // _BLK,),
        in_specs=[
            pl.BlockSpec((_BLK, _D), lambda i: (i, 0)),
            pl.BlockSpec((2, _BLK, _D), lambda i: (0, i, 0)),
            pl.BlockSpec((_D, _D), lambda i: (0, 0)),
            pl.BlockSpec((1, _D), lambda i: (0, 0)),
            pl.BlockSpec((_D, _D), lambda i: (0, 0)),
            pl.BlockSpec((1, _D), lambda i: (0, 0)),
            pl.BlockSpec((1, 1), lambda i: (0, 0)),
            pl.BlockSpec((1, _D), lambda i: (0, 0)),
            pl.BlockSpec((1, _D), lambda i: (0, 0)),
        ],
        out_specs=pl.BlockSpec((_BLK, _D), lambda i: (i, 0)),
        out_shape=jax.ShapeDtypeStruct((_N, _D), jnp.float32),
    )(h, agg, w1, b1.reshape(1, _D), w2, b2.reshape(1, _D),
      scale.reshape(1, 1), g.reshape(1, _D), bb.reshape(1, _D))


def kernel(x, edge_index, edge_attr, W_pre, b_pre, W1, b1, W2, b2, eps,
           ln_g, ln_b, W_head, b_head):
    src1 = edge_index[0]
    dst1 = edge_index[1]

    h = _tc_pre(x, W_pre, b_pre)
    for l in range(_L):
        agg = _mp_sc(h, edge_attr, src1, dst1)
        h = _tc_update(h, agg, W1[l], b1[l], W2[l], b2[l],
                       1.0 + eps[l], ln_g[l], ln_b[l])
    return _tc_head(h, W_head, b_head)


# speedup vs baseline: 7.6220x; 2.6239x over previous
import functools

import jax
import jax.numpy as jnp
from jax import lax
from jax.experimental import pallas as pl
from jax.experimental.pallas import tpu as pltpu
from jax.experimental.pallas import tpu_sc as plsc

_N = 10000
_E = 320000
_D = 128
_L = 3

_CH = 80
_NW = 32
_EPW = _E // _NW
_CHUNKS = _EPW // _CH
_NPAD = 10240
_ZROWS = _NPAD // 16
_ZCH = 32

_HIGH = jax.lax.Precision.HIGHEST


_NB = 4


def _mp_sc(h, edge_attr, src1, dst1):
    mesh = plsc.VectorSubcoreMesh(core_axis_name="c", subcore_axis_name="s")

    @functools.partial(
        pl.kernel,
        out_type=jax.ShapeDtypeStruct((2, _N, _D), jnp.float32),
        mesh=mesh,
        scratch_types=[
            pltpu.VMEM_SHARED((_NPAD, _D), jnp.float32),
            [pltpu.VMEM((_CH, _D), jnp.float32) for _ in range(_NB)],
            [pltpu.VMEM((_CH,), jnp.int32) for _ in range(2)],
            [pltpu.VMEM((_CH,), jnp.int32) for _ in range(_NB)],
            pltpu.VMEM((_ZCH, _D), jnp.float32),
            [pltpu.SemaphoreType.DMA for _ in range(_NB)],
            [pltpu.SemaphoreType.DMA for _ in range(_NB)],
            [pltpu.SemaphoreType.DMA for _ in range(_NB)],
            [pltpu.SemaphoreType.DMA for _ in range(2)],
            [pltpu.SemaphoreType.DMA for _ in range(_NB)],
        ],
    )
    def k(h_hbm, ea_hbm, src_hbm, dst_hbm, out_hbm,
          acc, rows, sv, dv, zbuf, gsem, esem, ssem, isv, idv):
        cid = lax.axis_index("c")
        sid = lax.axis_index("s")
        wid = cid * 16 + sid

        zero = jnp.zeros((16,), jnp.float32)

        @pl.loop(0, _ZCH)
        def _(i):
            for cc in range(_D // 16):
                zbuf[i, pl.ds(cc * 16, 16)] = zero

        for t in range(_ZROWS // _ZCH):
            pltpu.sync_copy(zbuf, acc.at[pl.ds(sid * _ZROWS + t * _ZCH, _ZCH)])
        plsc.subcore_barrier()

        def ebase(c):
            return wid * _EPW + c * _CH

        def drain_rows(sem, b):
            pltpu.make_async_copy(ea_hbm.at[pl.ds(0, _CH)], rows[b], sem).wait()

        def drain_idx(sem, buf):
            pltpu.make_async_copy(src_hbm.at[pl.ds(0, _CH)], buf, sem).wait()

        def compute(b):
            @pl.loop(0, _CH)
            def _(i):
                for cc in range(_D // 16):
                    sl = pl.ds(cc * 16, 16)
                    rows[b][i, sl] = jnp.maximum(rows[b][i, sl], 0.0)

        pltpu.async_copy(src_hbm.at[pl.ds(ebase(0), _CH)], sv[0], isv[0])
        pltpu.async_copy(src_hbm.at[pl.ds(ebase(1), _CH)], sv[1], isv[1])
        pltpu.async_copy(dst_hbm.at[pl.ds(ebase(0), _CH)], dv[0], idv[0])
        pltpu.async_copy(dst_hbm.at[pl.ds(ebase(1), _CH)], dv[1], idv[1])
        pltpu.async_copy(ea_hbm.at[pl.ds(ebase(0), _CH)], rows[0], esem[0])
        pltpu.async_copy(ea_hbm.at[pl.ds(ebase(1), _CH)], rows[1], esem[1])
        drain_idx(isv[0], sv[0])
        drain_rows(esem[0], 0)
        pltpu.async_copy(h_hbm.at[sv[0]], rows[0], gsem[0], add=True)

        def step(c, j, guarded):
            j1 = (j + 1) % _NB
            j2 = (j + 2) % _NB
            js = j % 2

            def refill():
                drain_rows(ssem[j2], j2)
                pltpu.async_copy(ea_hbm.at[pl.ds(ebase(c + 2), _CH)],
                                 rows[j2], esem[j2])

            def launch_gather():
                drain_idx(isv[(js + 1) % 2], sv[(js + 1) % 2])
                drain_rows(esem[j1], j1)
                pltpu.async_copy(h_hbm.at[sv[(js + 1) % 2]], rows[j1],
                                 gsem[j1], add=True)

            def prefetch_idx():
                pltpu.async_copy(src_hbm.at[pl.ds(ebase(c + 2), _CH)],
                                 sv[js], isv[js])
                pltpu.async_copy(dst_hbm.at[pl.ds(ebase(c + 2), _CH)],
                                 dv[j2], idv[j2])

            if guarded:
                if c >= 2:
                    drain_rows(ssem[j2], j2)
                if c + 2 < _CHUNKS:
                    pltpu.async_copy(ea_hbm.at[pl.ds(ebase(c + 2), _CH)],
                                     rows[j2], esem[j2])
                if c + 1 < _CHUNKS:
                    launch_gather()
            else:
                refill()
                launch_gather()
            drain_rows(gsem[j], j)
            drain_idx(idv[j], dv[j])
            compute(j)
            pltpu.async_copy(rows[j], acc.at[dv[j]], ssem[j], add=True)
            if not guarded or c + 2 < _CHUNKS:
                prefetch_idx()

        for j in range(_NB):
            step(j, j, guarded=True)

        @pl.loop(1, (_CHUNKS - 1) // _NB - 1)
        def _(tt):
            for j in range(_NB):
                step(tt * _NB + j, j, guarded=False)

        for j in range(_NB):
            step(120 + j, j, guarded=True)

        drain_rows(gsem[0], 0)
        drain_idx(idv[0], dv[0])
        compute(0)
        pltpu.sync_copy(rows[0], acc.at[dv[0]], add=True)
        for j in (2, 3):
            drain_rows(ssem[j], j)

        plsc.subcore_barrier()

        @pl.when(sid < 15)
        def _():
            pltpu.sync_copy(acc.at[pl.ds(sid * _ZROWS, _ZROWS)],
                            out_hbm.at[cid, pl.ds(sid * _ZROWS, _ZROWS)])

        @pl.when(sid == 15)
        def _():
            pltpu.sync_copy(acc.at[pl.ds(15 * _ZROWS, _N - 15 * _ZROWS)],
                            out_hbm.at[cid, pl.ds(15 * _ZROWS, _N - 15 * _ZROWS)])

    return k(h, edge_attr, src1, dst1)


_BLK = 400


def _pre_body(x_ref, w_ref, b_ref, o_ref):
    y = jnp.dot(x_ref[...], w_ref[...],
                preferred_element_type=jnp.float32, precision=_HIGH)
    o_ref[...] = jnp.maximum(y + b_ref[...], 0.0)


def _tc_pre(x, w, b):
    return pl.pallas_call(
        _pre_body,
        grid=(_N // _BLK,),
        in_specs=[
            pl.BlockSpec((_BLK, _D), lambda i: (i, 0)),
            pl.BlockSpec((_D, _D), lambda i: (0, 0)),
            pl.BlockSpec((1, _D), lambda i: (0, 0)),
        ],
        out_specs=pl.BlockSpec((_BLK, _D), lambda i: (i, 0)),
        out_shape=jax.ShapeDtypeStruct((_N, _D), jnp.float32),
    )(x, w, b.reshape(1, _D))


def _head_body(x_ref, w_ref, b_ref, o_ref):
    y = jnp.dot(x_ref[...], w_ref[...],
                preferred_element_type=jnp.float32, precision=_HIGH)
    o_ref[...] = y + b_ref[...]


def _tc_head(x, w, b):
    d_out = w.shape[1]
    return pl.pallas_call(
        _head_body,
        grid=(_N // _BLK,),
        in_specs=[
            pl.BlockSpec((_BLK, _D), lambda i: (i, 0)),
            pl.BlockSpec((_D, d_out), lambda i: (0, 0)),
            pl.BlockSpec((1, d_out), lambda i: (0, 0)),
        ],
        out_specs=pl.BlockSpec((_BLK, d_out), lambda i: (i, 0)),
        out_shape=jax.ShapeDtypeStruct((_N, d_out), jnp.float32),
    )(x, w, b.reshape(1, d_out))


def _upd_body(h_ref, a_ref, w1_ref, b1_ref, w2_ref, b2_ref, sc_ref,
              g_ref, bb_ref, o_ref):
    h = h_ref[...]
    agg = a_ref[0] + a_ref[1]
    z = sc_ref[0, 0] * h + agg
    z = jnp.maximum(
        jnp.dot(z, w1_ref[...], preferred_element_type=jnp.float32,
                precision=_HIGH) + b1_ref[...], 0.0)
    z = jnp.dot(z, w2_ref[...], preferred_element_type=jnp.float32,
                precision=_HIGH) + b2_ref[...]
    hn = h + z
    mu = jnp.mean(hn, axis=-1, keepdims=True)
    var = jnp.mean((hn - mu) ** 2, axis=-1, keepdims=True)
    o_ref[...] = g_ref[...] * (hn - mu) / jnp.sqrt(var + 1e-5) + bb_ref[...]


def _tc_update(h, agg, w1, b1, w2, b2, scale, g, bb):
    return pl.pallas_call(
        _upd_body,
        grid=(_N // _BLK,),
        in_specs=[
            pl.BlockSpec((_BLK, _D), lambda i: (i, 0)),
            pl.BlockSpec((2, _BLK, _D), lambda i: (0, i, 0)),
            pl.BlockSpec((_D, _D), lambda i: (0, 0)),
            pl.BlockSpec((1, _D), lambda i: (0, 0)),
            pl.BlockSpec((_D, _D), lambda i: (0, 0)),
            pl.BlockSpec((1, _D), lambda i: (0, 0)),
            pl.BlockSpec((1, 1), lambda i: (0, 0)),
            pl.BlockSpec((1, _D), lambda i: (0, 0)),
            pl.BlockSpec((1, _D), lambda i: (0, 0)),
        ],
        out_specs=pl.BlockSpec((_BLK, _D), lambda i: (i, 0)),
        out_shape=jax.ShapeDtypeStruct((_N, _D), jnp.float32),
    )(h, agg, w1, b1.reshape(1, _D), w2, b2.reshape(1, _D),
      scale.reshape(1, 1), g.reshape(1, _D), bb.reshape(1, _D))


def kernel(x, edge_index, edge_attr, W_pre, b_pre, W1, b1, W2, b2, eps,
           ln_g, ln_b, W_head, b_head):
    src1 = edge_index[0]
    dst1 = edge_index[1]

    h = _tc_pre(x, W_pre, b_pre)
    for l in range(_L):
        agg = _mp_sc(h, edge_attr, src1, dst1)
        h = _tc_update(h, agg, W1[l], b1[l], W2[l], b2[l],
                       1.0 + eps[l], ln_g[l], ln_b[l])
    return _tc_head(h, W_head, b_head)


# speedup vs baseline: 8.1119x; 1.0643x over previous
import functools

import jax
import jax.numpy as jnp
from jax import lax
from jax.experimental import pallas as pl
from jax.experimental.pallas import tpu as pltpu
from jax.experimental.pallas import tpu_sc as plsc

_N = 10000
_E = 320000
_D = 128
_L = 3

_CH = 80
_NW = 32
_EPW = _E // _NW
_CHUNKS = _EPW // _CH
_NPAD = 10240
_ZROWS = _NPAD // 16
_ZCH = 32

_HIGH = jax.lax.Precision.DEFAULT


_NB = 4


def _mp_sc(h, edge_attr, src1, dst1):
    mesh = plsc.VectorSubcoreMesh(core_axis_name="c", subcore_axis_name="s")

    @functools.partial(
        pl.kernel,
        out_type=jax.ShapeDtypeStruct((2, _N, _D), jnp.float32),
        mesh=mesh,
        scratch_types=[
            pltpu.VMEM_SHARED((_NPAD, _D), jnp.float32),
            [pltpu.VMEM((_CH, _D), jnp.float32) for _ in range(_NB)],
            [pltpu.VMEM((_CH,), jnp.int32) for _ in range(2)],
            [pltpu.VMEM((_CH,), jnp.int32) for _ in range(_NB)],
            pltpu.VMEM((_ZCH, _D), jnp.float32),
            [pltpu.SemaphoreType.DMA for _ in range(_NB)],
            [pltpu.SemaphoreType.DMA for _ in range(_NB)],
            [pltpu.SemaphoreType.DMA for _ in range(_NB)],
            [pltpu.SemaphoreType.DMA for _ in range(2)],
            [pltpu.SemaphoreType.DMA for _ in range(_NB)],
        ],
    )
    def k(h_hbm, ea_hbm, src_hbm, dst_hbm, out_hbm,
          acc, rows, sv, dv, zbuf, gsem, esem, ssem, isv, idv):
        cid = lax.axis_index("c")
        sid = lax.axis_index("s")
        wid = cid * 16 + sid

        zero = jnp.zeros((16,), jnp.float32)

        @pl.loop(0, _ZCH)
        def _(i):
            for cc in range(_D // 16):
                zbuf[i, pl.ds(cc * 16, 16)] = zero

        for t in range(_ZROWS // _ZCH):
            pltpu.sync_copy(zbuf, acc.at[pl.ds(sid * _ZROWS + t * _ZCH, _ZCH)])
        plsc.subcore_barrier()

        def ebase(c):
            return wid * _EPW + c * _CH

        def drain_rows(sem, b):
            pltpu.make_async_copy(ea_hbm.at[pl.ds(0, _CH)], rows[b], sem).wait()

        def drain_idx(sem, buf):
            pltpu.make_async_copy(src_hbm.at[pl.ds(0, _CH)], buf, sem).wait()

        def compute(b):
            @pl.loop(0, _CH)
            def _(i):
                for cc in range(_D // 16):
                    sl = pl.ds(cc * 16, 16)
                    rows[b][i, sl] = jnp.maximum(rows[b][i, sl], 0.0)

        pltpu.async_copy(src_hbm.at[pl.ds(ebase(0), _CH)], sv[0], isv[0])
        pltpu.async_copy(src_hbm.at[pl.ds(ebase(1), _CH)], sv[1], isv[1])
        pltpu.async_copy(dst_hbm.at[pl.ds(ebase(0), _CH)], dv[0], idv[0])
        pltpu.async_copy(dst_hbm.at[pl.ds(ebase(1), _CH)], dv[1], idv[1])
        pltpu.async_copy(ea_hbm.at[pl.ds(ebase(0), _CH)], rows[0], esem[0])
        pltpu.async_copy(ea_hbm.at[pl.ds(ebase(1), _CH)], rows[1], esem[1])
        drain_idx(isv[0], sv[0])
        drain_rows(esem[0], 0)
        pltpu.async_copy(h_hbm.at[sv[0]], rows[0], gsem[0], add=True)

        def step(c, j, guarded):
            j1 = (j + 1) % _NB
            j2 = (j + 2) % _NB
            js = j % 2

            def refill():
                drain_rows(ssem[j2], j2)
                pltpu.async_copy(ea_hbm.at[pl.ds(ebase(c + 2), _CH)],
                                 rows[j2], esem[j2])

            def launch_gather():
                drain_idx(isv[(js + 1) % 2], sv[(js + 1) % 2])
                drain_rows(esem[j1], j1)
                pltpu.async_copy(h_hbm.at[sv[(js + 1) % 2]], rows[j1],
                                 gsem[j1], add=True)

            def prefetch_idx():
                pltpu.async_copy(src_hbm.at[pl.ds(ebase(c + 2), _CH)],
                                 sv[js], isv[js])
                pltpu.async_copy(dst_hbm.at[pl.ds(ebase(c + 2), _CH)],
                                 dv[j2], idv[j2])

            if guarded:
                if c >= 2:
                    drain_rows(ssem[j2], j2)
                if c + 2 < _CHUNKS:
                    pltpu.async_copy(ea_hbm.at[pl.ds(ebase(c + 2), _CH)],
                                     rows[j2], esem[j2])
                if c + 1 < _CHUNKS:
                    launch_gather()
            else:
                refill()
                launch_gather()
            drain_rows(gsem[j], j)
            drain_idx(idv[j], dv[j])
            compute(j)
            pltpu.async_copy(rows[j], acc.at[dv[j]], ssem[j], add=True)
            if not guarded or c + 2 < _CHUNKS:
                prefetch_idx()

        for j in range(_NB):
            step(j, j, guarded=True)

        @pl.loop(1, (_CHUNKS - 1) // _NB - 1)
        def _(tt):
            for j in range(_NB):
                step(tt * _NB + j, j, guarded=False)

        for j in range(_NB):
            step(120 + j, j, guarded=True)

        drain_rows(gsem[0], 0)
        drain_idx(idv[0], dv[0])
        compute(0)
        pltpu.sync_copy(rows[0], acc.at[dv[0]], add=True)
        for j in (2, 3):
            drain_rows(ssem[j], j)

        plsc.subcore_barrier()

        @pl.when(sid < 15)
        def _():
            pltpu.sync_copy(acc.at[pl.ds(sid * _ZROWS, _ZROWS)],
                            out_hbm.at[cid, pl.ds(sid * _ZROWS, _ZROWS)])

        @pl.when(sid == 15)
        def _():
            pltpu.sync_copy(acc.at[pl.ds(15 * _ZROWS, _N - 15 * _ZROWS)],
                            out_hbm.at[cid, pl.ds(15 * _ZROWS, _N - 15 * _ZROWS)])

    return k(h, edge_attr, src1, dst1)


_BLK = 400


def _pre_body(x_ref, w_ref, b_ref, o_ref):
    y = jnp.dot(x_ref[...], w_ref[...],
                preferred_element_type=jnp.float32, precision=_HIGH)
    o_ref[...] = jnp.maximum(y + b_ref[...], 0.0)


def _tc_pre(x, w, b):
    return pl.pallas_call(
        _pre_body,
        grid=(_N // _BLK,),
        in_specs=[
            pl.BlockSpec((_BLK, _D), lambda i: (i, 0)),
            pl.BlockSpec((_D, _D), lambda i: (0, 0)),
            pl.BlockSpec((1, _D), lambda i: (0, 0)),
        ],
        out_specs=pl.BlockSpec((_BLK, _D), lambda i: (i, 0)),
        out_shape=jax.ShapeDtypeStruct((_N, _D), jnp.float32),
    )(x, w, b.reshape(1, _D))


def _head_body(x_ref, w_ref, b_ref, o_ref):
    y = jnp.dot(x_ref[...], w_ref[...],
                preferred_element_type=jnp.float32, precision=_HIGH)
    o_ref[...] = y + b_ref[...]


def _tc_head(x, w, b):
    d_out = w.shape[1]
    return pl.pallas_call(
        _head_body,
        grid=(_N // _BLK,),
        in_specs=[
            pl.BlockSpec((_BLK, _D), lambda i: (i, 0)),
            pl.BlockSpec((_D, d_out), lambda i: (0, 0)),
            pl.BlockSpec((1, d_out), lambda i: (0, 0)),
        ],
        out_specs=pl.BlockSpec((_BLK, d_out), lambda i: (i, 0)),
        out_shape=jax.ShapeDtypeStruct((_N, d_out), jnp.float32),
    )(x, w, b.reshape(1, d_out))


def _upd_body(h_ref, a_ref, w1_ref, b1_ref, w2_ref, b2_ref, sc_ref,
              g_ref, bb_ref, o_ref):
    h = h_ref[...]
    agg = a_ref[0] + a_ref[1]
    z = sc_ref[0, 0] * h + agg
    z = jnp.maximum(
        jnp.dot(z, w1_ref[...], preferred_element_type=jnp.float32,
                precision=_HIGH) + b1_ref[...], 0.0)
    z = jnp.dot(z, w2_ref[...], preferred_element_type=jnp.float32,
                precision=_HIGH) + b2_ref[...]
    hn = h + z
    mu = jnp.mean(hn, axis=-1, keepdims=True)
    var = jnp.mean((hn - mu) ** 2, axis=-1, keepdims=True)
    o_ref[...] = g_ref[...] * (hn - mu) / jnp.sqrt(var + 1e-5) + bb_ref[...]


def _upd_head_body(h_ref, a_ref, w1_ref, b1_ref, w2_ref, b2_ref, sc_ref,
                   g_ref, bb_ref, wh_ref, bh_ref, o_ref):
    h = h_ref[...]
    agg = a_ref[0] + a_ref[1]
    z = sc_ref[0, 0] * h + agg
    z = jnp.maximum(
        jnp.dot(z, w1_ref[...], preferred_element_type=jnp.float32,
                precision=_HIGH) + b1_ref[...], 0.0)
    z = jnp.dot(z, w2_ref[...], preferred_element_type=jnp.float32,
                precision=_HIGH) + b2_ref[...]
    hn = h + z
    mu = jnp.mean(hn, axis=-1, keepdims=True)
    var = jnp.mean((hn - mu) ** 2, axis=-1, keepdims=True)
    hl = g_ref[...] * (hn - mu) / jnp.sqrt(var + 1e-5) + bb_ref[...]
    o_ref[...] = jnp.dot(hl, wh_ref[...], preferred_element_type=jnp.float32,
                         precision=_HIGH) + bh_ref[...]


def _tc_update_head(h, agg, w1, b1, w2, b2, scale, g, bb, wh, bh):
    d_out = wh.shape[1]
    return pl.pallas_call(
        _upd_head_body,
        grid=(_N // _BLK,),
        in_specs=[
            pl.BlockSpec((_BLK, _D), lambda i: (i, 0)),
            pl.BlockSpec((2, _BLK, _D), lambda i: (0, i, 0)),
            pl.BlockSpec((_D, _D), lambda i: (0, 0)),
            pl.BlockSpec((1, _D), lambda i: (0, 0)),
            pl.BlockSpec((_D, _D), lambda i: (0, 0)),
            pl.BlockSpec((1, _D), lambda i: (0, 0)),
            pl.BlockSpec((1, 1), lambda i: (0, 0)),
            pl.BlockSpec((1, _D), lambda i: (0, 0)),
            pl.BlockSpec((1, _D), lambda i: (0, 0)),
            pl.BlockSpec((_D, d_out), lambda i: (0, 0)),
            pl.BlockSpec((1, d_out), lambda i: (0, 0)),
        ],
        out_specs=pl.BlockSpec((_BLK, d_out), lambda i: (i, 0)),
        out_shape=jax.ShapeDtypeStruct((_N, d_out), jnp.float32),
    )(h, agg, w1, b1.reshape(1, _D), w2, b2.reshape(1, _D),
      scale.reshape(1, 1), g.reshape(1, _D), bb.reshape(1, _D),
      wh, bh.reshape(1, d_out))


def _tc_update(h, agg, w1, b1, w2, b2, scale, g, bb):
    return pl.pallas_call(
        _upd_body,
        grid=(_N // _BLK,),
        in_specs=[
            pl.BlockSpec((_BLK, _D), lambda i: (i, 0)),
            pl.BlockSpec((2, _BLK, _D), lambda i: (0, i, 0)),
            pl.BlockSpec((_D, _D), lambda i: (0, 0)),
            pl.BlockSpec((1, _D), lambda i: (0, 0)),
            pl.BlockSpec((_D, _D), lambda i: (0, 0)),
            pl.BlockSpec((1, _D), lambda i: (0, 0)),
            pl.BlockSpec((1, 1), lambda i: (0, 0)),
            pl.BlockSpec((1, _D), lambda i: (0, 0)),
            pl.BlockSpec((1, _D), lambda i: (0, 0)),
        ],
        out_specs=pl.BlockSpec((_BLK, _D), lambda i: (i, 0)),
        out_shape=jax.ShapeDtypeStruct((_N, _D), jnp.float32),
    )(h, agg, w1, b1.reshape(1, _D), w2, b2.reshape(1, _D),
      scale.reshape(1, 1), g.reshape(1, _D), bb.reshape(1, _D))


def kernel(x, edge_index, edge_attr, W_pre, b_pre, W1, b1, W2, b2, eps,
           ln_g, ln_b, W_head, b_head):
    src1 = edge_index[0]
    dst1 = edge_index[1]

    h = _tc_pre(x, W_pre, b_pre)
    for l in range(_L - 1):
        agg = _mp_sc(h, edge_attr, src1, dst1)
        h = _tc_update(h, agg, W1[l], b1[l], W2[l], b2[l],
                       1.0 + eps[l], ln_g[l], ln_b[l])
    agg = _mp_sc(h, edge_attr, src1, dst1)
    return _tc_update_head(h, agg, W1[_L - 1], b1[_L - 1], W2[_L - 1],
                           b2[_L - 1], 1.0 + eps[_L - 1], ln_g[_L - 1],
                           ln_b[_L - 1], W_head, b_head)
